# baseline (speedup 1.0000x reference)
import functools

import jax
import jax.numpy as jnp
from jax import lax
from jax.experimental import pallas as pl
from jax.experimental.pallas import tpu as pltpu
from jax.experimental.pallas import tpu_sc as plsc

N = 10000
E = 320000
D = 128
G = 256

NC = 2
NS = 16
NW = NC * NS

NPAD = 10240
TRASH_N = NPAD - 1

EPT = E // NW
ECH = 80
ECHX = ECH + 2

NPT = NPAD // NS


def _tile_id():
    return lax.axis_index("c") * NS + lax.axis_index("s")


def _deg_body(dst_idx, ones16, z16, deg_out, idx_v, ones_v, deg_sh):
    c = lax.axis_index("c")
    s = lax.axis_index("s")
    w = _tile_id()
    pltpu.sync_copy(z16.at[pl.ds(s * NPT, NPT)], deg_sh.at[pl.ds(s * NPT, NPT)])
    pltpu.sync_copy(ones16, ones_v)
    pltpu.sync_copy(dst_idx.at[w], idx_v)
    plsc.subcore_barrier()

    def chunk(i, carry):
        pltpu.sync_copy(ones_v, deg_sh.at[idx_v.at[i]], add=True)
        return carry

    lax.fori_loop(0, ECH, chunk, 0)
    plsc.subcore_barrier()
    pltpu.sync_copy(deg_sh.at[pl.ds(s * NPT, NPT)],
                    deg_out.at[c, pl.ds(s * NPT, NPT)])


def _sc_degrees(dst_idx, ones16, z16):
    mesh = plsc.VectorSubcoreMesh(core_axis_name="c", subcore_axis_name="s")
    fn = pl.kernel(
        _deg_body,
        out_type=jax.ShapeDtypeStruct((NC, NPAD, 16), jnp.float32),
        mesh=mesh,
        scratch_types=[
            pltpu.VMEM((ECHX, 128), jnp.int32),
            pltpu.VMEM((128, 16), jnp.float32),
            pltpu.VMEM_SHARED((NPAD, 16), jnp.float32),
        ],
        compiler_params=pltpu.CompilerParams(use_tc_tiling_on_sc=False),
    )
    return fn(dst_idx, ones16, z16)


def _gs_body(src_idx, dst_idx, table, zrows, agg_out,
             sidx_v, didx_v, rows_v, acc_sh, sem0, sem1):
    c = lax.axis_index("c")
    s = lax.axis_index("s")
    w = _tile_id()
    pltpu.sync_copy(zrows.at[pl.ds(s * NPT, NPT)],
                    acc_sh.at[pl.ds(s * NPT, NPT)])
    pltpu.sync_copy(src_idx.at[w], sidx_v)
    pltpu.sync_copy(dst_idx.at[w], didx_v)
    plsc.subcore_barrier()

    pltpu.async_copy(table.at[sidx_v.at[0]], rows_v.at[0], sem0)
    pltpu.async_copy(table.at[sidx_v.at[1]], rows_v.at[1], sem1)

    def pair(k, carry):
        i = 2 * k
        pltpu.make_async_copy(table.at[sidx_v.at[i]], rows_v.at[0],
                              sem0).wait()
        pltpu.sync_copy(rows_v.at[0], acc_sh.at[didx_v.at[i]], add=True)
        pltpu.async_copy(table.at[sidx_v.at[i + 2]], rows_v.at[0], sem0)
        pltpu.make_async_copy(table.at[sidx_v.at[i + 1]], rows_v.at[1],
                              sem1).wait()
        pltpu.sync_copy(rows_v.at[1], acc_sh.at[didx_v.at[i + 1]], add=True)
        pltpu.async_copy(table.at[sidx_v.at[i + 3]], rows_v.at[1], sem1)
        return carry

    lax.fori_loop(0, ECH // 2, pair, 0)
    pltpu.make_async_copy(table.at[sidx_v.at[ECH]], rows_v.at[0],
                          sem0).wait()
    pltpu.make_async_copy(table.at[sidx_v.at[ECH + 1]], rows_v.at[1],
                          sem1).wait()
    plsc.subcore_barrier()
    pltpu.sync_copy(acc_sh.at[pl.ds(s * NPT, NPT)],
                    agg_out.at[c, pl.ds(s * NPT, NPT)])


def _sc_gather_scatter(src_idx, dst_idx, table, zrows):
    mesh = plsc.VectorSubcoreMesh(core_axis_name="c", subcore_axis_name="s")
    fn = pl.kernel(
        _gs_body,
        out_type=jax.ShapeDtypeStruct((NC, NPAD, 64), jnp.float32),
        mesh=mesh,
        scratch_types=[
            pltpu.VMEM((ECHX, 128), jnp.int32),
            pltpu.VMEM((ECHX, 128), jnp.int32),
            pltpu.VMEM((2, 128, 64), jnp.float32),
            pltpu.VMEM_SHARED((NPAD, 64), jnp.float32),
            pltpu.SemaphoreType.DMA,
            pltpu.SemaphoreType.DMA,
        ],
        compiler_params=pltpu.CompilerParams(use_tc_tiling_on_sc=False),
    )
    return fn(src_idx, dst_idx, table, zrows)


def _dinv_block(degb):
    deg = degb[0, :, 0] + degb[1, :, 0] + 1.0
    return lax.rsqrt(deg)[:, None]


def _tc1_body(xb, w1, degb, ob):
    h = jnp.dot(xb[...], w1[...], preferred_element_type=jnp.float32)
    ob[...] = h * _dinv_block(degb[...])


def _tc_pre(x_pad, W1, deg_part):
    return pl.pallas_call(
        _tc1_body,
        grid=(NPAD // 256,),
        in_specs=[
            pl.BlockSpec((256, D), lambda i: (i, 0)),
            pl.BlockSpec((D, 64), lambda i: (0, 0)),
            pl.BlockSpec((NC, 256, 16), lambda i: (0, i, 0)),
        ],
        out_specs=pl.BlockSpec((256, 64), lambda i: (i, 0)),
        out_shape=jax.ShapeDtypeStruct((NPAD, 64), jnp.float32),
    )(x_pad, W1, deg_part)


def _tc2_body(aggb, hsb, degb, b1, w2, ob):
    dinv = _dinv_block(degb[...])
    h1 = jnp.maximum((aggb[0] + aggb[1] + hsb[...]) * dinv + b1[...], 0.0)
    ob[...] = jnp.dot(h1, w2[...], preferred_element_type=jnp.float32) * dinv


def _tc_mid(agg1, hs1, deg_part, b1r, W2):
    return pl.pallas_call(
        _tc2_body,
        grid=(NPAD // 256,),
        in_specs=[
            pl.BlockSpec((NC, 256, 64), lambda i: (0, i, 0)),
            pl.BlockSpec((256, 64), lambda i: (i, 0)),
            pl.BlockSpec((NC, 256, 16), lambda i: (0, i, 0)),
            pl.BlockSpec((1, 64), lambda i: (0, 0)),
            pl.BlockSpec((64, 64), lambda i: (0, 0)),
        ],
        out_specs=pl.BlockSpec((256, 64), lambda i: (i, 0)),
        out_shape=jax.ShapeDtypeStruct((NPAD, 64), jnp.float32),
    )(agg1, hs1, deg_part, b1r, W2)


def _tc3_body(aggb, hsb, degb, b2, batchb, sums_ob, cnts_ob):
    i = pl.program_id(0)
    dinv = _dinv_block(degb[...])
    h2 = jnp.maximum((aggb[0] + aggb[1] + hsb[...]) * dinv + b2[...], 0.0)
    gids = lax.broadcasted_iota(jnp.int32, (G, 256), 0)
    oh = (gids == batchb[0]).astype(jnp.float32)

    @pl.when(i == 0)
    def _():
        sums_ob[...] = jnp.zeros_like(sums_ob)
        cnts_ob[...] = jnp.zeros_like(cnts_ob)

    sums_ob[...] += jnp.dot(oh, h2, preferred_element_type=jnp.float32)
    cnts_ob[...] += jnp.sum(oh, axis=1, keepdims=True)


def _tc_post_pool(agg2, hs2, deg_part, b2r, batch2d):
    return pl.pallas_call(
        _tc3_body,
        grid=(NPAD // 256,),
        in_specs=[
            pl.BlockSpec((NC, 256, 64), lambda i: (0, i, 0)),
            pl.BlockSpec((256, 64), lambda i: (i, 0)),
            pl.BlockSpec((NC, 256, 16), lambda i: (0, i, 0)),
            pl.BlockSpec((1, 64), lambda i: (0, 0)),
            pl.BlockSpec((1, 1, 256), lambda i: (i, 0, 0)),
        ],
        out_specs=[
            pl.BlockSpec((G, 64), lambda i: (0, 0)),
            pl.BlockSpec((G, 1), lambda i: (0, 0)),
        ],
        out_shape=[
            jax.ShapeDtypeStruct((G, 64), jnp.float32),
            jax.ShapeDtypeStruct((G, 1), jnp.float32),
        ],
    )(agg2, hs2, deg_part, b2r, batch2d)


def _tc4_body(sumb, cntb, wf1, bf1, wf2, bf2, ob):
    emb = sumb[...] / jnp.maximum(cntb[...], 1.0)
    h = jnp.maximum(
        jnp.dot(emb, wf1[...], preferred_element_type=jnp.float32) + bf1[...],
        0.0)
    z = jnp.dot(h, wf2[...], preferred_element_type=jnp.float32) + bf2[...]
    ob[...] = 1.0 / (1.0 + jnp.exp(-z))


def _tc_head(sums, cnts, Wf1p, bf1p, Wf2p, bf2p):
    return pl.pallas_call(
        _tc4_body,
        grid=(1,),
        in_specs=[
            pl.BlockSpec((G, 64), lambda i: (0, 0)),
            pl.BlockSpec((G, 1), lambda i: (0, 0)),
            pl.BlockSpec((64, 128), lambda i: (0, 0)),
            pl.BlockSpec((1, 128), lambda i: (0, 0)),
            pl.BlockSpec((128, 128), lambda i: (0, 0)),
            pl.BlockSpec((1, 128), lambda i: (0, 0)),
        ],
        out_specs=pl.BlockSpec((G, 128), lambda i: (0, 0)),
        out_shape=jax.ShapeDtypeStruct((G, 128), jnp.float32),
    )(sums, cnts, Wf1p, bf1p, Wf2p, bf2p)


def kernel(x, edge_index, batch, W1, b1, W2, b2, Wf1, bf1, Wf2, bf2):
    i32 = jnp.int32
    src2 = edge_index[0].astype(i32).reshape(NW, EPT)
    dst2 = edge_index[1].astype(i32).reshape(NW, EPT)
    pad = jnp.full((NW, ECHX * 128 - EPT), TRASH_N, dtype=i32)
    src_idx = jnp.concatenate([src2, pad], axis=1).reshape(NW, ECHX, 128)
    dst_idx = jnp.concatenate([dst2, pad], axis=1).reshape(NW, ECHX, 128)

    z16 = jnp.zeros((NPAD, 16), jnp.float32)
    z64 = jnp.zeros((NPAD, 64), jnp.float32)
    ones16 = jnp.ones((128, 16), jnp.float32)
    x_pad = jnp.concatenate(
        [x, jnp.zeros((NPAD - N, D), jnp.float32)], axis=0)
    batch2d = jnp.concatenate(
        [batch.astype(i32), jnp.full((NPAD - N,), G, dtype=i32)]
    ).reshape(NPAD // 256, 1, 256)

    deg_part = _sc_degrees(dst_idx, ones16, z16)

    hs1 = _tc_pre(x_pad, W1, deg_part)
    agg1 = _sc_gather_scatter(src_idx, dst_idx, hs1, z64)
    hs2 = _tc_mid(agg1, hs1, deg_part, b1.reshape(1, 64), W2)
    agg2 = _sc_gather_scatter(src_idx, dst_idx, hs2, z64)
    sums, cnts = _tc_post_pool(agg2, hs2, deg_part, b2.reshape(1, 64),
                               batch2d)

    Wf1p = jnp.zeros((64, 128), jnp.float32).at[:, :32].set(Wf1)
    bf1p = jnp.zeros((1, 128), jnp.float32).at[0, :32].set(bf1)
    Wf2p = jnp.zeros((128, 128), jnp.float32).at[:32, :1].set(Wf2)
    bf2p = jnp.zeros((1, 128), jnp.float32).at[0, 0].set(bf2[0])
    out = _tc_head(sums, cnts, Wf1p, bf1p, Wf2p, bf2p)
    return out[:, 0]


# speedup vs baseline: 1.3119x; 1.3119x over previous
import functools

import jax
import jax.numpy as jnp
from jax import lax
from jax.experimental import pallas as pl
from jax.experimental.pallas import tpu as pltpu
from jax.experimental.pallas import tpu_sc as plsc

N = 10000
E = 320000
D = 128
G = 256

NC = 2
NS = 16
NW = NC * NS

NPAD = 10240
TRASH_N = NPAD - 1

EPT = E // NW
ECH = 80
ECHX = ECH + 2

NPT = NPAD // NS


def _tile_id():
    return lax.axis_index("c") * NS + lax.axis_index("s")


def _deg_body(dst_idx, ones16, z16, deg_out, idx_v, ones_v, deg_sh):
    c = lax.axis_index("c")
    s = lax.axis_index("s")
    w = _tile_id()
    pltpu.sync_copy(z16.at[pl.ds(s * NPT, NPT)], deg_sh.at[pl.ds(s * NPT, NPT)])
    pltpu.sync_copy(ones16, ones_v)
    pltpu.sync_copy(dst_idx.at[w], idx_v)
    plsc.subcore_barrier()

    def chunk(i, carry):
        pltpu.sync_copy(ones_v, deg_sh.at[idx_v.at[i]], add=True)
        return carry

    lax.fori_loop(0, ECH, chunk, 0)
    plsc.subcore_barrier()
    pltpu.sync_copy(deg_sh.at[pl.ds(s * NPT, NPT)],
                    deg_out.at[c, pl.ds(s * NPT, NPT)])


def _sc_degrees(dst_idx, ones16, z16):
    mesh = plsc.VectorSubcoreMesh(core_axis_name="c", subcore_axis_name="s")
    fn = pl.kernel(
        _deg_body,
        out_type=jax.ShapeDtypeStruct((NC, NPAD, 16), jnp.float32),
        mesh=mesh,
        scratch_types=[
            pltpu.VMEM((ECHX, 128), jnp.int32),
            pltpu.VMEM((128, 16), jnp.float32),
            pltpu.VMEM_SHARED((NPAD, 16), jnp.float32),
        ],
        compiler_params=pltpu.CompilerParams(use_tc_tiling_on_sc=False),
    )
    return fn(dst_idx, ones16, z16)


def _gs_body(src_idx, dst_idx, table, zrows, agg_out,
             sidx_v, didx_v, rows_v, acc_sh, sem0, sem1):
    c = lax.axis_index("c")
    s = lax.axis_index("s")
    w = _tile_id()
    pltpu.sync_copy(zrows.at[pl.ds(s * NPT, NPT)],
                    acc_sh.at[pl.ds(s * NPT, NPT)])
    pltpu.sync_copy(src_idx.at[w], sidx_v)
    pltpu.sync_copy(dst_idx.at[w], didx_v)
    plsc.subcore_barrier()

    def chunk(i, carry):
        pltpu.async_copy(table.at[sidx_v.at[i]], rows_v.at[0], sem0).wait()
        pltpu.sync_copy(rows_v.at[0], acc_sh.at[didx_v.at[i]], add=True)
        return carry

    lax.fori_loop(0, ECH, chunk, 0)
    plsc.subcore_barrier()
    pltpu.sync_copy(acc_sh.at[pl.ds(s * NPT, NPT)],
                    agg_out.at[c, pl.ds(s * NPT, NPT)])


def _sc_gather_scatter(src_idx, dst_idx, table, zrows):
    mesh = plsc.VectorSubcoreMesh(core_axis_name="c", subcore_axis_name="s")
    fn = pl.kernel(
        _gs_body,
        out_type=jax.ShapeDtypeStruct((NC, NPAD, 64), jnp.float32),
        mesh=mesh,
        scratch_types=[
            pltpu.VMEM((ECHX, 128), jnp.int32),
            pltpu.VMEM((ECHX, 128), jnp.int32),
            pltpu.VMEM((2, 128, 64), jnp.float32),
            pltpu.VMEM_SHARED((NPAD, 64), jnp.float32),
            pltpu.SemaphoreType.DMA,
            pltpu.SemaphoreType.DMA,
        ],
        compiler_params=pltpu.CompilerParams(use_tc_tiling_on_sc=False),
    )
    return fn(src_idx, dst_idx, table, zrows)


def _dinv_block(degb):
    deg = degb[0, :, 0] + degb[1, :, 0] + 1.0
    return lax.rsqrt(deg)[:, None]


def _tc1_body(xb, w1, degb, ob):
    h = jnp.dot(xb[...], w1[...], preferred_element_type=jnp.float32)
    ob[...] = h * _dinv_block(degb[...])


def _tc_pre(x_pad, W1, deg_part):
    return pl.pallas_call(
        _tc1_body,
        grid=(NPAD // 256,),
        in_specs=[
            pl.BlockSpec((256, D), lambda i: (i, 0)),
            pl.BlockSpec((D, 64), lambda i: (0, 0)),
            pl.BlockSpec((NC, 256, 16), lambda i: (0, i, 0)),
        ],
        out_specs=pl.BlockSpec((256, 64), lambda i: (i, 0)),
        out_shape=jax.ShapeDtypeStruct((NPAD, 64), jnp.float32),
    )(x_pad, W1, deg_part)


def _tc2_body(aggb, hsb, degb, b1, w2, ob):
    dinv = _dinv_block(degb[...])
    h1 = jnp.maximum((aggb[0] + aggb[1] + hsb[...]) * dinv + b1[...], 0.0)
    ob[...] = jnp.dot(h1, w2[...], preferred_element_type=jnp.float32) * dinv


def _tc_mid(agg1, hs1, deg_part, b1r, W2):
    return pl.pallas_call(
        _tc2_body,
        grid=(NPAD // 256,),
        in_specs=[
            pl.BlockSpec((NC, 256, 64), lambda i: (0, i, 0)),
            pl.BlockSpec((256, 64), lambda i: (i, 0)),
            pl.BlockSpec((NC, 256, 16), lambda i: (0, i, 0)),
            pl.BlockSpec((1, 64), lambda i: (0, 0)),
            pl.BlockSpec((64, 64), lambda i: (0, 0)),
        ],
        out_specs=pl.BlockSpec((256, 64), lambda i: (i, 0)),
        out_shape=jax.ShapeDtypeStruct((NPAD, 64), jnp.float32),
    )(agg1, hs1, deg_part, b1r, W2)


def _tc3_body(aggb, hsb, degb, b2, batchb, sums_ob, cnts_ob):
    i = pl.program_id(0)
    dinv = _dinv_block(degb[...])
    h2 = jnp.maximum((aggb[0] + aggb[1] + hsb[...]) * dinv + b2[...], 0.0)
    gids = lax.broadcasted_iota(jnp.int32, (G, 256), 0)
    oh = (gids == batchb[0]).astype(jnp.float32)

    @pl.when(i == 0)
    def _():
        sums_ob[...] = jnp.zeros_like(sums_ob)
        cnts_ob[...] = jnp.zeros_like(cnts_ob)

    sums_ob[...] += jnp.dot(oh, h2, preferred_element_type=jnp.float32)
    cnts_ob[...] += jnp.sum(oh, axis=1, keepdims=True)


def _tc_post_pool(agg2, hs2, deg_part, b2r, batch2d):
    return pl.pallas_call(
        _tc3_body,
        grid=(NPAD // 256,),
        in_specs=[
            pl.BlockSpec((NC, 256, 64), lambda i: (0, i, 0)),
            pl.BlockSpec((256, 64), lambda i: (i, 0)),
            pl.BlockSpec((NC, 256, 16), lambda i: (0, i, 0)),
            pl.BlockSpec((1, 64), lambda i: (0, 0)),
            pl.BlockSpec((1, 1, 256), lambda i: (i, 0, 0)),
        ],
        out_specs=[
            pl.BlockSpec((G, 64), lambda i: (0, 0)),
            pl.BlockSpec((G, 1), lambda i: (0, 0)),
        ],
        out_shape=[
            jax.ShapeDtypeStruct((G, 64), jnp.float32),
            jax.ShapeDtypeStruct((G, 1), jnp.float32),
        ],
    )(agg2, hs2, deg_part, b2r, batch2d)


def _tc4_body(sumb, cntb, wf1, bf1, wf2, bf2, ob):
    emb = sumb[...] / jnp.maximum(cntb[...], 1.0)
    h = jnp.maximum(
        jnp.dot(emb, wf1[...], preferred_element_type=jnp.float32) + bf1[...],
        0.0)
    z = jnp.dot(h, wf2[...], preferred_element_type=jnp.float32) + bf2[...]
    ob[...] = 1.0 / (1.0 + jnp.exp(-z))


def _tc_head(sums, cnts, Wf1p, bf1p, Wf2p, bf2p):
    return pl.pallas_call(
        _tc4_body,
        grid=(1,),
        in_specs=[
            pl.BlockSpec((G, 64), lambda i: (0, 0)),
            pl.BlockSpec((G, 1), lambda i: (0, 0)),
            pl.BlockSpec((64, 128), lambda i: (0, 0)),
            pl.BlockSpec((1, 128), lambda i: (0, 0)),
            pl.BlockSpec((128, 128), lambda i: (0, 0)),
            pl.BlockSpec((1, 128), lambda i: (0, 0)),
        ],
        out_specs=pl.BlockSpec((G, 128), lambda i: (0, 0)),
        out_shape=jax.ShapeDtypeStruct((G, 128), jnp.float32),
    )(sums, cnts, Wf1p, bf1p, Wf2p, bf2p)


def kernel(x, edge_index, batch, W1, b1, W2, b2, Wf1, bf1, Wf2, bf2):
    i32 = jnp.int32
    src2 = edge_index[0].astype(i32).reshape(NW, EPT)
    dst2 = edge_index[1].astype(i32).reshape(NW, EPT)
    pad = jnp.full((NW, ECHX * 128 - EPT), TRASH_N, dtype=i32)
    src_idx = jnp.concatenate([src2, pad], axis=1).reshape(NW, ECHX, 128)
    dst_idx = jnp.concatenate([dst2, pad], axis=1).reshape(NW, ECHX, 128)

    z16 = jnp.zeros((NPAD, 16), jnp.float32)
    z64 = jnp.zeros((NPAD, 64), jnp.float32)
    ones16 = jnp.ones((128, 16), jnp.float32)
    x_pad = jnp.concatenate(
        [x, jnp.zeros((NPAD - N, D), jnp.float32)], axis=0)
    batch2d = jnp.concatenate(
        [batch.astype(i32), jnp.full((NPAD - N,), G, dtype=i32)]
    ).reshape(NPAD // 256, 1, 256)

    deg_part = _sc_degrees(dst_idx, ones16, z16)

    hs1 = _tc_pre(x_pad, W1, deg_part)
    agg1 = _sc_gather_scatter(src_idx, dst_idx, hs1, z64)
    hs2 = _tc_mid(agg1, hs1, deg_part, b1.reshape(1, 64), W2)
    agg2 = _sc_gather_scatter(src_idx, dst_idx, hs2, z64)
    sums, cnts = _tc_post_pool(agg2, hs2, deg_part, b2.reshape(1, 64),
                               batch2d)

    Wf1p = jnp.zeros((64, 128), jnp.float32).at[:, :32].set(Wf1)
    bf1p = jnp.zeros((1, 128), jnp.float32).at[0, :32].set(bf1)
    Wf2p = jnp.zeros((128, 128), jnp.float32).at[:32, :1].set(Wf2)
    bf2p = jnp.zeros((1, 128), jnp.float32).at[0, 0].set(bf2[0])
    out = _tc_head(sums, cnts, Wf1p, bf1p, Wf2p, bf2p)
    return out[:, 0]


# speedup vs baseline: 1.3121x; 1.0001x over previous
import functools

import jax
import jax.numpy as jnp
from jax import lax
from jax.experimental import pallas as pl
from jax.experimental.pallas import tpu as pltpu
from jax.experimental.pallas import tpu_sc as plsc

N = 10000
E = 320000
D = 128
G = 256

NC = 2
NS = 16
NW = NC * NS

NPAD = 10240
TRASH_N = NPAD - 1

EPT = E // NW
ECH = 80
ECHX = ECH + 2

NPT = NPAD // NS


def _tile_id():
    return lax.axis_index("c") * NS + lax.axis_index("s")


def _deg_body(dst_idx, ones16, z16, deg_out, idx_v, ones_v, deg_sh):
    c = lax.axis_index("c")
    s = lax.axis_index("s")
    w = _tile_id()
    pltpu.sync_copy(z16.at[pl.ds(s * NPT, NPT)], deg_sh.at[pl.ds(s * NPT, NPT)])
    pltpu.sync_copy(ones16, ones_v)
    pltpu.sync_copy(dst_idx.at[w], idx_v)
    plsc.subcore_barrier()

    def chunk(i, carry):
        pltpu.sync_copy(ones_v, deg_sh.at[idx_v.at[i]], add=True)
        return carry

    lax.fori_loop(0, ECH, chunk, 0)
    plsc.subcore_barrier()
    pltpu.sync_copy(deg_sh.at[pl.ds(s * NPT, NPT)],
                    deg_out.at[c, pl.ds(s * NPT, NPT)])


def _sc_degrees(dst_idx, ones16, z16):
    mesh = plsc.VectorSubcoreMesh(core_axis_name="c", subcore_axis_name="s")
    fn = pl.kernel(
        _deg_body,
        out_type=jax.ShapeDtypeStruct((NC, NPAD, 16), jnp.float32),
        mesh=mesh,
        scratch_types=[
            pltpu.VMEM((ECHX, 128), jnp.int32),
            pltpu.VMEM((128, 16), jnp.float32),
            pltpu.VMEM_SHARED((NPAD, 16), jnp.float32),
        ],
        compiler_params=pltpu.CompilerParams(use_tc_tiling_on_sc=False),
    )
    return fn(dst_idx, ones16, z16)


def _gs_body(src_idx, dst_idx, table, zrows, agg_out,
             sidx_v, didx_v, rows_v, acc_sh, sem0):
    c = lax.axis_index("c")
    s = lax.axis_index("s")
    w = _tile_id()
    pltpu.sync_copy(zrows.at[pl.ds(s * NPT, NPT)],
                    acc_sh.at[pl.ds(s * NPT, NPT)])
    pltpu.sync_copy(src_idx.at[w], sidx_v)
    pltpu.sync_copy(dst_idx.at[w], didx_v)
    plsc.subcore_barrier()

    def chunk(i, carry):
        pltpu.async_copy(table.at[sidx_v.at[i]], rows_v, sem0).wait()
        pltpu.sync_copy(rows_v, acc_sh.at[didx_v.at[i]], add=True)
        return carry

    lax.fori_loop(0, ECH, chunk, 0)
    plsc.subcore_barrier()
    pltpu.sync_copy(acc_sh.at[pl.ds(s * NPT, NPT)],
                    agg_out.at[c, pl.ds(s * NPT, NPT)])


def _sc_gather_scatter(src_idx, dst_idx, table, zrows):
    mesh = plsc.VectorSubcoreMesh(core_axis_name="c", subcore_axis_name="s")
    fn = pl.kernel(
        _gs_body,
        out_type=jax.ShapeDtypeStruct((NC, NPAD, 64), jnp.float32),
        mesh=mesh,
        scratch_types=[
            pltpu.VMEM((ECHX, 128), jnp.int32),
            pltpu.VMEM((ECHX, 128), jnp.int32),
            pltpu.VMEM((128, 64), jnp.float32),
            pltpu.VMEM_SHARED((NPAD, 64), jnp.float32),
            pltpu.SemaphoreType.DMA,
        ],
        compiler_params=pltpu.CompilerParams(use_tc_tiling_on_sc=False),
    )
    return fn(src_idx, dst_idx, table, zrows)


def _dinv_block(degb):
    deg = degb[0, :, 0] + degb[1, :, 0] + 1.0
    return lax.rsqrt(deg)[:, None]


def _tc1_body(xb, w1, degb, ob):
    h = jnp.dot(xb[...], w1[...], preferred_element_type=jnp.float32)
    ob[...] = h * _dinv_block(degb[...])


def _tc_pre(x_pad, W1, deg_part):
    return pl.pallas_call(
        _tc1_body,
        grid=(NPAD // 256,),
        in_specs=[
            pl.BlockSpec((256, D), lambda i: (i, 0)),
            pl.BlockSpec((D, 64), lambda i: (0, 0)),
            pl.BlockSpec((NC, 256, 16), lambda i: (0, i, 0)),
        ],
        out_specs=pl.BlockSpec((256, 64), lambda i: (i, 0)),
        out_shape=jax.ShapeDtypeStruct((NPAD, 64), jnp.float32),
    )(x_pad, W1, deg_part)


def _tc2_body(aggb, hsb, degb, b1, w2, ob):
    dinv = _dinv_block(degb[...])
    h1 = jnp.maximum((aggb[0] + aggb[1] + hsb[...]) * dinv + b1[...], 0.0)
    ob[...] = jnp.dot(h1, w2[...], preferred_element_type=jnp.float32) * dinv


def _tc_mid(agg1, hs1, deg_part, b1r, W2):
    return pl.pallas_call(
        _tc2_body,
        grid=(NPAD // 256,),
        in_specs=[
            pl.BlockSpec((NC, 256, 64), lambda i: (0, i, 0)),
            pl.BlockSpec((256, 64), lambda i: (i, 0)),
            pl.BlockSpec((NC, 256, 16), lambda i: (0, i, 0)),
            pl.BlockSpec((1, 64), lambda i: (0, 0)),
            pl.BlockSpec((64, 64), lambda i: (0, 0)),
        ],
        out_specs=pl.BlockSpec((256, 64), lambda i: (i, 0)),
        out_shape=jax.ShapeDtypeStruct((NPAD, 64), jnp.float32),
    )(agg1, hs1, deg_part, b1r, W2)


def _tc3_body(aggb, hsb, degb, b2, batchb, sums_ob, cnts_ob):
    i = pl.program_id(0)
    dinv = _dinv_block(degb[...])
    h2 = jnp.maximum((aggb[0] + aggb[1] + hsb[...]) * dinv + b2[...], 0.0)
    gids = lax.broadcasted_iota(jnp.int32, (G, 256), 0)
    oh = (gids == batchb[0]).astype(jnp.float32)

    @pl.when(i == 0)
    def _():
        sums_ob[...] = jnp.zeros_like(sums_ob)
        cnts_ob[...] = jnp.zeros_like(cnts_ob)

    sums_ob[...] += jnp.dot(oh, h2, preferred_element_type=jnp.float32)
    cnts_ob[...] += jnp.sum(oh, axis=1, keepdims=True)


def _tc_post_pool(agg2, hs2, deg_part, b2r, batch2d):
    return pl.pallas_call(
        _tc3_body,
        grid=(NPAD // 256,),
        in_specs=[
            pl.BlockSpec((NC, 256, 64), lambda i: (0, i, 0)),
            pl.BlockSpec((256, 64), lambda i: (i, 0)),
            pl.BlockSpec((NC, 256, 16), lambda i: (0, i, 0)),
            pl.BlockSpec((1, 64), lambda i: (0, 0)),
            pl.BlockSpec((1, 1, 256), lambda i: (i, 0, 0)),
        ],
        out_specs=[
            pl.BlockSpec((G, 64), lambda i: (0, 0)),
            pl.BlockSpec((G, 1), lambda i: (0, 0)),
        ],
        out_shape=[
            jax.ShapeDtypeStruct((G, 64), jnp.float32),
            jax.ShapeDtypeStruct((G, 1), jnp.float32),
        ],
    )(agg2, hs2, deg_part, b2r, batch2d)


def _tc4_body(sumb, cntb, wf1, bf1, wf2, bf2, ob):
    emb = sumb[...] / jnp.maximum(cntb[...], 1.0)
    h = jnp.maximum(
        jnp.dot(emb, wf1[...], preferred_element_type=jnp.float32) + bf1[...],
        0.0)
    z = jnp.dot(h, wf2[...], preferred_element_type=jnp.float32) + bf2[...]
    ob[...] = 1.0 / (1.0 + jnp.exp(-z))


def _tc_head(sums, cnts, Wf1p, bf1p, Wf2p, bf2p):
    return pl.pallas_call(
        _tc4_body,
        grid=(1,),
        in_specs=[
            pl.BlockSpec((G, 64), lambda i: (0, 0)),
            pl.BlockSpec((G, 1), lambda i: (0, 0)),
            pl.BlockSpec((64, 128), lambda i: (0, 0)),
            pl.BlockSpec((1, 128), lambda i: (0, 0)),
            pl.BlockSpec((128, 128), lambda i: (0, 0)),
            pl.BlockSpec((1, 128), lambda i: (0, 0)),
        ],
        out_specs=pl.BlockSpec((G, 128), lambda i: (0, 0)),
        out_shape=jax.ShapeDtypeStruct((G, 128), jnp.float32),
    )(sums, cnts, Wf1p, bf1p, Wf2p, bf2p)


def kernel(x, edge_index, batch, W1, b1, W2, b2, Wf1, bf1, Wf2, bf2):
    i32 = jnp.int32
    src2 = edge_index[0].astype(i32).reshape(NW, EPT)
    dst2 = edge_index[1].astype(i32).reshape(NW, EPT)
    pad = jnp.full((NW, ECHX * 128 - EPT), TRASH_N, dtype=i32)
    src_idx = jnp.concatenate([src2, pad], axis=1).reshape(NW, ECHX, 128)
    dst_idx = jnp.concatenate([dst2, pad], axis=1).reshape(NW, ECHX, 128)

    z16 = jnp.zeros((NPAD, 16), jnp.float32)
    z64 = jnp.zeros((NPAD, 64), jnp.float32)
    ones16 = jnp.ones((128, 16), jnp.float32)
    x_pad = jnp.concatenate(
        [x, jnp.zeros((NPAD - N, D), jnp.float32)], axis=0)
    batch2d = jnp.concatenate(
        [batch.astype(i32), jnp.full((NPAD - N,), G, dtype=i32)]
    ).reshape(NPAD // 256, 1, 256)

    deg_part = _sc_degrees(dst_idx, ones16, z16)

    hs1 = _tc_pre(x_pad, W1, deg_part)
    agg1 = _sc_gather_scatter(src_idx, dst_idx, hs1, z64)
    hs2 = _tc_mid(agg1, hs1, deg_part, b1.reshape(1, 64), W2)
    agg2 = _sc_gather_scatter(src_idx, dst_idx, hs2, z64)
    sums, cnts = _tc_post_pool(agg2, hs2, deg_part, b2.reshape(1, 64),
                               batch2d)

    Wf1p = jnp.zeros((64, 128), jnp.float32).at[:, :32].set(Wf1)
    bf1p = jnp.zeros((1, 128), jnp.float32).at[0, :32].set(bf1)
    Wf2p = jnp.zeros((128, 128), jnp.float32).at[:32, :1].set(Wf2)
    bf2p = jnp.zeros((1, 128), jnp.float32).at[0, 0].set(bf2[0])
    out = _tc_head(sums, cnts, Wf1p, bf1p, Wf2p, bf2p)
    return out[:, 0]
